# baseline (speedup 1.0000x reference)
import jax
import jax.numpy as jnp
from jax import lax
from jax.experimental import pallas as pl
from jax.experimental.pallas import tpu as pltpu
from jax.experimental.pallas import tpu_sc as plsc

_DIM = 64
_B = 16384
_GAMMA = 12.0
_EMB_RANGE = (12.0 + 2.0) / _DIM
_PI = 3.14
_SCALE = _EMB_RANGE / _PI
_HALF_INV_SCALE = 1.0 / (2.0 * _SCALE)
_NREL = 1000

_NC = 2
_NS = 16
_NW = _NC * _NS
_BPW = _B // _NW
_CHUNK = 64
_NCHUNK = _BPW // _CHUNK
_REC = 2 * _DIM

_TC_BLK = 2048
_TC_NB = _B // _TC_BLK


def _sc_body(idx_t_hbm, ent_hbm, rel_hbm, de_out, part_out,
             si_all, pi_all, oi_all,
             s0, s1, p0, p1, o0, o1, de0, de1, part_v,
             isem, gsem0, gsem1, wsem0, wsem1):
    s_v = (s0, s1)
    p_v = (p0, p1)
    o_v = (o0, o1)
    de_v = (de0, de1)
    gsem = (gsem0, gsem1)
    wsem = (wsem0, wsem1)

    wid = lax.axis_index("s") * _NC + lax.axis_index("c")
    base = wid * _BPW

    i0 = pltpu.async_copy(idx_t_hbm.at[pl.ds(base, _BPW)], si_all, isem)
    i1 = pltpu.async_copy(idx_t_hbm.at[pl.ds(_B + base, _BPW)], pi_all, isem)
    i2 = pltpu.async_copy(idx_t_hbm.at[pl.ds(2 * _B + base, _BPW)], oi_all,
                          isem)
    i0.wait()
    i1.wait()
    i2.wait()

    def fire(ci, buf):
        sl = pl.ds(ci * _CHUNK, _CHUNK)
        return (
            pltpu.async_copy(ent_hbm.at[si_all.at[sl]], s_v[buf], gsem[buf]),
            pltpu.async_copy(rel_hbm.at[pi_all.at[sl]], p_v[buf], gsem[buf]),
            pltpu.async_copy(ent_hbm.at[oi_all.at[sl]], o_v[buf], gsem[buf]),
        )

    def compute(buf, acc0):
        sv, pv, ov, dv = s_v[buf], p_v[buf], o_v[buf], de_v[buf]

        def unpk(r, woff):
            w = pv[r, pl.ds(woff, 16)]
            a = lax.bitcast_convert_type(w << 16, jnp.float32)
            b = lax.bitcast_convert_type(w & jnp.int32(-65536), jnp.float32)
            return a, b

        @plsc.parallel_loop(0, _CHUNK, unroll=2, carry=acc0)
        def row(r, acc):
            for q in range(2):
                pp = unpk(r, 16 * q)
                av = unpk(r, 32 + 16 * q)
                bv = unpk(r, 64 + 16 * q)
                for h in range(2):
                    c = pl.ds(32 * q + 16 * h, 16)
                    m = pl.ds(_DIM + 32 * q + 16 * h, 16)
                    dv[r, c] = ((sv[r, c] + pp[h]) - ov[r, c]) * \
                        _HALF_INV_SCALE
                    e = sv[r, m] * av[h] - jnp.abs(ov[r, m]) * bv[h]
                    acc = acc + e * e
            return acc

        return row

    ghandles = {0: fire(0, 0)}
    whandles = {}
    acc = jnp.zeros((16,), jnp.float32)
    for ci in range(_NCHUNK):
        buf = ci & 1
        if ci + 1 < _NCHUNK:
            ghandles[ci + 1] = fire(ci + 1, 1 - buf)
        for h in ghandles.pop(ci):
            h.wait()
        if ci >= 2:
            whandles.pop(ci - 2).wait()
        acc = compute(buf, acc)
        whandles[ci] = pltpu.async_copy(
            de_v[buf], de_out.at[pl.ds(base + ci * _CHUNK, _CHUNK), :],
            wsem[buf])
    part_v[...] = acc
    pltpu.sync_copy(part_v, part_out.at[pl.ds(wid * 16, 16)])
    for ci in (_NCHUNK - 2, _NCHUNK - 1):
        whandles.pop(ci).wait()


_sc_gather_fused = pl.kernel(
    _sc_body,
    out_type=(
        jax.ShapeDtypeStruct((_B, _DIM), jnp.float32),
        jax.ShapeDtypeStruct((_NW * 16,), jnp.float32),
    ),
    mesh=plsc.VectorSubcoreMesh(core_axis_name="c", subcore_axis_name="s"),
    scratch_types=[
        pltpu.VMEM((_BPW,), jnp.int32),
        pltpu.VMEM((_BPW,), jnp.int32),
        pltpu.VMEM((_BPW,), jnp.int32),
        pltpu.VMEM((_CHUNK, 2 * _DIM), jnp.float32),
        pltpu.VMEM((_CHUNK, 2 * _DIM), jnp.float32),
        pltpu.VMEM((_CHUNK, 2 * _DIM), jnp.int32),
        pltpu.VMEM((_CHUNK, 2 * _DIM), jnp.int32),
        pltpu.VMEM((_CHUNK, 2 * _DIM), jnp.float32),
        pltpu.VMEM((_CHUNK, 2 * _DIM), jnp.float32),
        pltpu.VMEM((_CHUNK, _DIM), jnp.float32),
        pltpu.VMEM((_CHUNK, _DIM), jnp.float32),
        pltpu.VMEM((16,), jnp.float32),
        pltpu.SemaphoreType.DMA,
        pltpu.SemaphoreType.DMA,
        pltpu.SemaphoreType.DMA,
        pltpu.SemaphoreType.DMA,
        pltpu.SemaphoreType.DMA,
    ],
)


_INV_PI = 0.3183098861837907
_PI_A = 3.140625
_PI_B = 9.67653589793e-4
_S1 = -1.6666654611e-1
_S2 = 8.3321608736e-3
_S3 = -1.9515295891e-4


def _sin_sq(d):
    t = d * _INV_PI
    half = jnp.where(t >= 0.0, 0.5, -0.5)
    k = (t + half).astype(jnp.int32).astype(jnp.float32)
    r = (d - k * _PI_A) - k * _PI_B
    r2 = r * r
    sr = r * (1.0 + r2 * (_S1 + r2 * (_S2 + r2 * _S3)))
    return sr * sr


def _tc_score_body(d_ref, part_ref, out_ref, psq_ref):
    i = pl.program_id(0)
    psq_ref[pl.ds(i * _TC_BLK, _TC_BLK)] = jnp.sum(_sin_sq(d_ref[...]),
                                                   axis=1)

    @pl.when(i == _TC_NB - 1)
    def _():
        mod_term = jnp.sqrt(jnp.sum(part_ref[...]))
        out_ref[...] = (_GAMMA - mod_term) - 0.5 * jnp.sqrt(psq_ref[...])


def _tc_score(d, part):
    return pl.pallas_call(
        _tc_score_body,
        grid=(_TC_NB,),
        in_specs=[
            pl.BlockSpec((_TC_BLK, _DIM), lambda i: (i, 0)),
            pl.BlockSpec((_NW * 16,), lambda i: (0,)),
        ],
        out_specs=pl.BlockSpec((_B,), lambda i: (0,)),
        out_shape=jax.ShapeDtypeStruct((_B,), jnp.float32),
        scratch_shapes=[
            pltpu.VMEM((_B,), jnp.float32),
        ],
    )(d, part)


def _pack_cols(x):
    xb = lax.bitcast_convert_type(x.astype(jnp.bfloat16),
                                  jnp.uint16).astype(jnp.uint32)
    lo = jnp.concatenate([xb[:, 0:16], xb[:, 32:48]], axis=1)
    hi = jnp.concatenate([xb[:, 16:32], xb[:, 48:64]], axis=1)
    return lax.bitcast_convert_type(lo | (hi << 16), jnp.int32)


def kernel(inputs, ent_table, rel_table):
    phase_p = rel_table[:, :_DIM]
    mod_p = rel_table[:, _DIM:2 * _DIM]
    bias_p = rel_table[:, 2 * _DIM:]
    nap = -jnp.abs(mod_p)
    bias_c = jnp.minimum(bias_p, 1.0)
    bias_c = jnp.where(bias_c < nap, nap, bias_c)
    rel_packed = jnp.concatenate(
        [_pack_cols(phase_p), _pack_cols(mod_p + bias_c),
         _pack_cols(1.0 - bias_c),
         jnp.zeros((_NREL, 32), jnp.int32)], axis=1)
    idx_flat = inputs.T.reshape(-1)
    d, part = _sc_gather_fused(idx_flat, ent_table, rel_packed)
    return _tc_score(d, part).reshape(_B, 1)


# speedup vs baseline: 1.0050x; 1.0050x over previous
import jax
import jax.numpy as jnp
from jax import lax
from jax.experimental import pallas as pl
from jax.experimental.pallas import tpu as pltpu
from jax.experimental.pallas import tpu_sc as plsc

_DIM = 64
_B = 16384
_GAMMA = 12.0
_EMB_RANGE = (12.0 + 2.0) / _DIM
_PI = 3.14
_SCALE = _EMB_RANGE / _PI
_HALF_INV_SCALE = 1.0 / (2.0 * _SCALE)
_NREL = 1000

_NC = 2
_NS = 16
_NW = _NC * _NS
_BPW = _B // _NW
_CHUNK = 64
_NCHUNK = _BPW // _CHUNK

_TC_BLK = 2048
_TC_NB = _B // _TC_BLK


def _sc_body(idx_t_hbm, ent_hbm, rel_hbm, de_out, part_out,
             si_all, pi_all, oi_all,
             s0, s1, p0, p1, o0, o1, de0, de1, part_v,
             isem, gsem0, gsem1, wsem0, wsem1):
    s_v = (s0, s1)
    p_v = (p0, p1)
    o_v = (o0, o1)
    de_v = (de0, de1)
    gsem = (gsem0, gsem1)
    wsem = (wsem0, wsem1)

    wid = lax.axis_index("s") * _NC + lax.axis_index("c")
    base = wid * _BPW

    i0 = pltpu.async_copy(idx_t_hbm.at[pl.ds(base, _BPW)], si_all, isem)
    i1 = pltpu.async_copy(idx_t_hbm.at[pl.ds(_B + base, _BPW)], pi_all, isem)
    i2 = pltpu.async_copy(idx_t_hbm.at[pl.ds(2 * _B + base, _BPW)], oi_all,
                          isem)
    i0.wait()
    i1.wait()
    i2.wait()

    def fire(ci, buf):
        sl = pl.ds(ci * _CHUNK, _CHUNK)
        return (
            pltpu.async_copy(ent_hbm.at[si_all.at[sl]], s_v[buf], gsem[buf]),
            pltpu.async_copy(rel_hbm.at[pi_all.at[sl]], p_v[buf], gsem[buf]),
            pltpu.async_copy(ent_hbm.at[oi_all.at[sl]], o_v[buf], gsem[buf]),
        )

    def compute(buf, acc0):
        sv, pv, ov, dv = s_v[buf], p_v[buf], o_v[buf], de_v[buf]

        def unpk(r, woff):
            w = pv[r, pl.ds(woff, 16)]
            a = lax.bitcast_convert_type(w << 16, jnp.float32)
            b = lax.bitcast_convert_type(w & jnp.int32(-65536), jnp.float32)
            return a, b

        @plsc.parallel_loop(0, _CHUNK, unroll=4, carry=acc0)
        def row(r, acc):
            for q in range(2):
                pp = unpk(r, 16 * q)
                av = unpk(r, 32 + 16 * q)
                bv = unpk(r, 64 + 16 * q)
                for h in range(2):
                    c = pl.ds(32 * q + 16 * h, 16)
                    m = pl.ds(_DIM + 32 * q + 16 * h, 16)
                    dv[r, c] = ((sv[r, c] + pp[h]) - ov[r, c]) * \
                        _HALF_INV_SCALE
                    e = sv[r, m] * av[h] - jnp.abs(ov[r, m]) * bv[h]
                    acc = acc + e * e
            return acc

        return row

    ghandles = {0: fire(0, 0)}
    whandles = {}
    acc = jnp.zeros((16,), jnp.float32)
    for ci in range(_NCHUNK):
        buf = ci & 1
        if ci + 1 < _NCHUNK:
            ghandles[ci + 1] = fire(ci + 1, 1 - buf)
        for h in ghandles.pop(ci):
            h.wait()
        if ci >= 2:
            whandles.pop(ci - 2).wait()
        acc = compute(buf, acc)
        whandles[ci] = pltpu.async_copy(
            de_v[buf], de_out.at[pl.ds(base + ci * _CHUNK, _CHUNK), :],
            wsem[buf])
    part_v[...] = acc
    pltpu.sync_copy(part_v, part_out.at[pl.ds(wid * 16, 16)])
    for ci in (_NCHUNK - 2, _NCHUNK - 1):
        whandles.pop(ci).wait()


_sc_gather_fused = pl.kernel(
    _sc_body,
    out_type=(
        jax.ShapeDtypeStruct((_B, _DIM), jnp.float32),
        jax.ShapeDtypeStruct((_NW * 16,), jnp.float32),
    ),
    mesh=plsc.VectorSubcoreMesh(core_axis_name="c", subcore_axis_name="s"),
    scratch_types=[
        pltpu.VMEM((_BPW,), jnp.int32),
        pltpu.VMEM((_BPW,), jnp.int32),
        pltpu.VMEM((_BPW,), jnp.int32),
        pltpu.VMEM((_CHUNK, 2 * _DIM), jnp.float32),
        pltpu.VMEM((_CHUNK, 2 * _DIM), jnp.float32),
        pltpu.VMEM((_CHUNK, 2 * _DIM), jnp.int32),
        pltpu.VMEM((_CHUNK, 2 * _DIM), jnp.int32),
        pltpu.VMEM((_CHUNK, 2 * _DIM), jnp.float32),
        pltpu.VMEM((_CHUNK, 2 * _DIM), jnp.float32),
        pltpu.VMEM((_CHUNK, _DIM), jnp.float32),
        pltpu.VMEM((_CHUNK, _DIM), jnp.float32),
        pltpu.VMEM((16,), jnp.float32),
        pltpu.SemaphoreType.DMA,
        pltpu.SemaphoreType.DMA,
        pltpu.SemaphoreType.DMA,
        pltpu.SemaphoreType.DMA,
        pltpu.SemaphoreType.DMA,
    ],
)


_INV_PI = 0.3183098861837907
_PI_A = 3.140625
_PI_B = 9.67653589793e-4
_S1 = -1.6666654611e-1
_S2 = 8.3321608736e-3
_S3 = -1.9515295891e-4


def _sin_sq(d):
    t = d * _INV_PI
    half = jnp.where(t >= 0.0, 0.5, -0.5)
    k = (t + half).astype(jnp.int32).astype(jnp.float32)
    r = (d - k * _PI_A) - k * _PI_B
    r2 = r * r
    sr = r * (1.0 + r2 * (_S1 + r2 * (_S2 + r2 * _S3)))
    return sr * sr


def _tc_score_body(d_ref, part_ref, out_ref, psq_ref):
    i = pl.program_id(0)
    psq_ref[pl.ds(i * _TC_BLK, _TC_BLK)] = jnp.sum(_sin_sq(d_ref[...]),
                                                   axis=1)

    @pl.when(i == _TC_NB - 1)
    def _():
        mod_term = jnp.sqrt(jnp.sum(part_ref[...]))
        out_ref[...] = (_GAMMA - mod_term) - 0.5 * jnp.sqrt(psq_ref[...])


def _tc_score(d, part):
    return pl.pallas_call(
        _tc_score_body,
        grid=(_TC_NB,),
        in_specs=[
            pl.BlockSpec((_TC_BLK, _DIM), lambda i: (i, 0)),
            pl.BlockSpec((_NW * 16,), lambda i: (0,)),
        ],
        out_specs=pl.BlockSpec((_B,), lambda i: (0,)),
        out_shape=jax.ShapeDtypeStruct((_B,), jnp.float32),
        scratch_shapes=[
            pltpu.VMEM((_B,), jnp.float32),
        ],
    )(d, part)


def _pack_cols(x):
    xb = lax.bitcast_convert_type(x.astype(jnp.bfloat16),
                                  jnp.uint16).astype(jnp.uint32)
    lo = jnp.concatenate([xb[:, 0:16], xb[:, 32:48]], axis=1)
    hi = jnp.concatenate([xb[:, 16:32], xb[:, 48:64]], axis=1)
    return lax.bitcast_convert_type(lo | (hi << 16), jnp.int32)


def kernel(inputs, ent_table, rel_table):
    phase_p = rel_table[:, :_DIM]
    mod_p = rel_table[:, _DIM:2 * _DIM]
    bias_p = rel_table[:, 2 * _DIM:]
    nap = -jnp.abs(mod_p)
    bias_c = jnp.minimum(bias_p, 1.0)
    bias_c = jnp.where(bias_c < nap, nap, bias_c)
    rel_packed = jnp.concatenate(
        [_pack_cols(phase_p), _pack_cols(mod_p + bias_c),
         _pack_cols(1.0 - bias_c),
         jnp.zeros((_NREL, 32), jnp.int32)], axis=1)
    idx_flat = inputs.T.reshape(-1)
    d, part = _sc_gather_fused(idx_flat, ent_table, rel_packed)
    return _tc_score(d, part).reshape(_B, 1)
